# baseline (speedup 1.0000x reference)
import functools

import jax
import jax.numpy as jnp
from jax import lax
from jax.experimental import pallas as pl
from jax.experimental.pallas import tpu as pltpu
from jax.experimental.pallas import tpu_sc as plsc

LANES = 16
BATCH = 24


def _sqrt_group(ss):
  iv = [lax.bitcast_convert_type(s, jnp.int32) for s in ss]
  iv = [jnp.int32(0x5F3759DF) - lax.shift_right_logical(i, 1) for i in iv]
  ys = [lax.bitcast_convert_type(i, jnp.float32) for i in iv]
  hs = [s * jnp.float32(0.5) for s in ss]
  for _ in range(1):
    ts = [y * y for y in ys]
    ts = [h * t for h, t in zip(hs, ts)]
    ts = [jnp.float32(1.5) - t for t in ts]
    ys = [y * t for y, t in zip(ys, ts)]
  return [s * y for s, y in zip(ss, ys)]


def _unpack_lo(v):
  return lax.bitcast_convert_type(lax.shift_left(v, 16), jnp.float32)


def _unpack_hi(v):
  return lax.bitcast_convert_type(v, jnp.float32)


@functools.partial(jax.jit, static_argnames=("n_atoms", "row32", "n_geoms",
                                             "n_edges", "nbatch", "per_worker"))
def _run(xp, bflat, *, n_atoms, row32, n_geoms, n_edges, nbatch, per_worker):
  info = plsc.get_sparse_core_info()
  nc = info.num_cores
  wpd = row32 // 3
  n_pairs = n_geoms // 32
  pair_group = 8
  nhalf = nbatch // 2

  mesh = plsc.VectorSubcoreMesh(core_axis_name="c", subcore_axis_name="s")

  @functools.partial(
      pl.kernel,
      mesh=mesh,
      out_type=jax.ShapeDtypeStruct((n_edges, n_geoms), jnp.float32),
      scratch_types=[
          pltpu.VMEM((2 * per_worker,), jnp.int32),
          pltpu.VMEM((2 * BATCH, row32), jnp.int32),
          pltpu.VMEM((2 * BATCH, row32), jnp.int32),
          pltpu.VMEM((BATCH, n_geoms), jnp.float32),
          pltpu.VMEM((BATCH, n_geoms), jnp.float32),
          pltpu.SemaphoreType.DMA,
          pltpu.SemaphoreType.DMA,
          pltpu.SemaphoreType.DMA,
          pltpu.SemaphoreType.DMA,
      ],
  )
  def k(x_hbm, b_hbm, out_hbm, idx_all, rows0, rows1, out0, out1,
        gsem0, gsem1, osem0, osem1):
    rows = (rows0, rows1)
    outs = (out0, out1)
    gsems = (gsem0, gsem1)
    osems = (osem0, osem1)

    wid = lax.axis_index("s") * nc + lax.axis_index("c")
    wbase = wid * per_worker

    def gather_cp(kb, sl):
      off = pl.multiple_of(kb * (2 * BATCH), 2 * BATCH)
      idx_sl = idx_all.at[pl.ds(off, 2 * BATCH)]
      return pltpu.make_async_copy(x_hbm.at[idx_sl], rows[sl], gsems[sl])

    def store_cp(base, sl):
      return pltpu.make_async_copy(
          outs[sl], out_hbm.at[pl.ds(base, BATCH), :], osems[sl])

    def compute_batch(rv, ov):
      group = 6

      def pair_body(p, _):
        colw = pl.multiple_of(p * LANES, LANES)
        colo = pl.multiple_of(p * (2 * LANES), 2 * LANES)

        def flush(state):
          es, rr = state
          for e, rlo, rhi in zip(es, rr[0::2], rr[1::2]):
            ov[e, pl.ds(colo, LANES)] = rlo
            ov[e, pl.ds(colo + LANES, LANES)] = rhi

        prev = None
        for e0 in range(0, BATCH, group):
          es = list(range(e0, e0 + group))
          ss = []
          for e in es:
            dlo = []
            dhi = []
            for d in range(3):
              vi = rv[2 * e, pl.ds(colw + d * wpd, LANES)]
              vj = rv[2 * e + 1, pl.ds(colw + d * wpd, LANES)]
              dlo.append(_unpack_lo(vi) - _unpack_lo(vj))
              dhi.append(_unpack_hi(vi) - _unpack_hi(vj))
            ss.append(dlo[0] * dlo[0] + dlo[1] * dlo[1] + dlo[2] * dlo[2])
            ss.append(dhi[0] * dhi[0] + dhi[1] * dhi[1] + dhi[2] * dhi[2])
          if prev is not None:
            flush(prev)
          prev = (es, _sqrt_group(ss))
        flush(prev)
        return 0

      lax.fori_loop(0, n_pairs, pair_body, 0, unroll=2)

    any_live = wbase < n_edges

    @pl.when(any_live)
    def _():
      pltpu.sync_copy(b_hbm.at[pl.ds(2 * wbase, 2 * per_worker)], idx_all)
      gather_cp(0, 0).start()

    def pair_body(kk, _):
      for sub in (0, 1):
        kb = 2 * kk + sub
        base = wbase + kb * BATCH
        live = base < n_edges

        @pl.when(live)
        def _(kb=kb, base=base, sl=sub, kk=kk):
          next_ok = (wbase + (kb + 1) * BATCH) < n_edges
          if sl == 1:
            next_ok = jnp.logical_and(next_ok, kk < nhalf - 1)

          @pl.when(next_ok)
          def _():
            gather_cp(kb + 1, 1 - sl).start()

          @pl.when(kb >= 2)
          def _():
            store_cp(base - 2 * BATCH, sl).wait()

          gather_cp(kb, sl).wait()

          compute_batch(rows[sl], outs[sl])

          full = base + BATCH <= n_edges

          @pl.when(full)
          def _():
            store_cp(base, sl).start()

          @pl.when(jnp.logical_not(full))
          def _():
            def tail_body(e, _):
              @pl.when(base + e < n_edges)
              def _():
                pltpu.sync_copy(outs[sl].at[e], out_hbm.at[base + e])
              return 0

            lax.fori_loop(0, BATCH, tail_body, 0, unroll=False)

      return 0

    lax.fori_loop(0, nhalf, pair_body, 0, unroll=False)

    n_my = jnp.maximum(jnp.int32(0),
                       jnp.minimum(jnp.int32(n_edges) - wbase,
                                   jnp.int32(per_worker)))
    n_live = (n_my + BATCH - 1) // BATCH
    n_full = n_my // BATCH
    for kq in (2, 1):
      kp = n_live - kq

      @pl.when(jnp.logical_and(kp >= 0, kp < n_full))
      def _(kp=kp):
        sl = lax.rem(kp, jnp.int32(2))

        @pl.when(sl == 0)
        def _():
          store_cp(wbase + kp * BATCH, 0).wait()

        @pl.when(sl == 1)
        def _():
          store_cp(wbase + kp * BATCH, 1).wait()

  return k(xp, bflat)


def kernel(input, bonds):
  n_atoms, three, n_geoms = input.shape
  n_edges = bonds.shape[0]
  row32 = three * n_geoms // 2

  n_workers = 32
  per_worker_edges = -(-n_edges // n_workers)
  nbatch = -(-per_worker_edges // BATCH)
  nbatch += nbatch % 2
  per_worker = nbatch * BATCH
  e_pad = n_workers * per_worker

  xs = input.reshape(n_atoms, three, n_geoms // 32, 2, LANES)
  xs = jnp.swapaxes(xs, 3, 4).astype(jnp.bfloat16)
  xp = lax.bitcast_convert_type(xs, jnp.int32).reshape(n_atoms, row32)

  bflat = jnp.concatenate(
      [bonds.reshape(-1), jnp.zeros(2 * (e_pad - n_edges), jnp.int32)])

  return _run(xp, bflat, n_atoms=n_atoms, row32=row32, n_geoms=n_geoms,
              n_edges=n_edges, nbatch=nbatch, per_worker=per_worker)


# speedup vs baseline: 1.0957x; 1.0957x over previous
import functools

import jax
import jax.numpy as jnp
from jax import lax
from jax.experimental import pallas as pl
from jax.experimental.pallas import tpu as pltpu
from jax.experimental.pallas import tpu_sc as plsc

LANES = 16
BATCH = 24


def _sqrt_group(ss):
  iv = [lax.bitcast_convert_type(s, jnp.int32) for s in ss]
  iv = [jnp.int32(0x5F3759DF) - lax.shift_right_logical(i, 1) for i in iv]
  ys = [lax.bitcast_convert_type(i, jnp.float32) for i in iv]
  hs = [s * jnp.float32(0.5) for s in ss]
  for _ in range(1):
    ts = [y * y for y in ys]
    ts = [h * t for h, t in zip(hs, ts)]
    ts = [jnp.float32(1.5) - t for t in ts]
    ys = [y * t for y, t in zip(ys, ts)]
  return [s * y for s, y in zip(ss, ys)]


def _unpack_lo(v):
  return lax.bitcast_convert_type(lax.shift_left(v, 16), jnp.float32)


def _unpack_hi(v):
  return lax.bitcast_convert_type(v, jnp.float32)


@functools.partial(jax.jit, static_argnames=("n_atoms", "row32", "n_geoms",
                                             "n_edges", "nbatch", "per_worker"))
def _run(xp, bflat, *, n_atoms, row32, n_geoms, n_edges, nbatch, per_worker):
  info = plsc.get_sparse_core_info()
  nc = info.num_cores
  wpd = row32 // 3
  n_pairs = n_geoms // 32
  pair_group = 8
  nhalf = nbatch // 2

  mesh = plsc.VectorSubcoreMesh(core_axis_name="c", subcore_axis_name="s")

  @functools.partial(
      pl.kernel,
      mesh=mesh,
      out_type=jax.ShapeDtypeStruct((n_edges, n_geoms), jnp.float32),
      scratch_types=[
          pltpu.VMEM((2 * per_worker,), jnp.int32),
          pltpu.VMEM((2 * BATCH, row32), jnp.int32),
          pltpu.VMEM((2 * BATCH, row32), jnp.int32),
          pltpu.VMEM((BATCH, n_geoms), jnp.float32),
          pltpu.VMEM((BATCH, n_geoms), jnp.float32),
          pltpu.SemaphoreType.DMA,
          pltpu.SemaphoreType.DMA,
          pltpu.SemaphoreType.DMA,
          pltpu.SemaphoreType.DMA,
      ],
  )
  def k(x_hbm, b_hbm, out_hbm, idx_all, rows0, rows1, out0, out1,
        gsem0, gsem1, osem0, osem1):
    rows = (rows0, rows1)
    outs = (out0, out1)
    gsems = (gsem0, gsem1)
    osems = (osem0, osem1)

    wid = lax.axis_index("s") * nc + lax.axis_index("c")
    wbase = wid * per_worker

    def gather_cp(kb, sl):
      off = pl.multiple_of(kb * (2 * BATCH), 2 * BATCH)
      idx_sl = idx_all.at[pl.ds(off, 2 * BATCH)]
      return pltpu.make_async_copy(x_hbm.at[idx_sl], rows[sl], gsems[sl])

    def store_cp(base, sl):
      return pltpu.make_async_copy(
          outs[sl], out_hbm.at[pl.ds(base, BATCH), :], osems[sl])

    def compute_batch(rv, ov):
      group = 6

      def pair_body(p, _):
        colw = pl.multiple_of(p * LANES, LANES)
        colo = pl.multiple_of(p * (2 * LANES), 2 * LANES)

        def flush(state):
          es, rr = state
          for e, rlo, rhi in zip(es, rr[0::2], rr[1::2]):
            ov[e, pl.ds(colo, LANES)] = rlo
            ov[e, pl.ds(colo + LANES, LANES)] = rhi

        prev = None
        for e0 in range(0, BATCH, group):
          es = list(range(e0, e0 + group))
          ss = []
          for e in es:
            dlo = []
            dhi = []
            for d in range(3):
              vi = rv[2 * e, pl.ds(colw + d * wpd, LANES)]
              vj = rv[2 * e + 1, pl.ds(colw + d * wpd, LANES)]
              dlo.append(_unpack_lo(vi) - _unpack_lo(vj))
              dhi.append(_unpack_hi(vi) - _unpack_hi(vj))
            ss.append(dlo[0] * dlo[0] + dlo[1] * dlo[1] + dlo[2] * dlo[2])
            ss.append(dhi[0] * dhi[0] + dhi[1] * dhi[1] + dhi[2] * dhi[2])
          if prev is not None:
            flush(prev)
          prev = (es, _sqrt_group(ss))
        flush(prev)
        return 0

      lax.fori_loop(0, n_pairs, pair_body, 0, unroll=False)

    any_live = wbase < n_edges

    @pl.when(any_live)
    def _():
      pltpu.sync_copy(b_hbm.at[pl.ds(2 * wbase, 2 * per_worker)], idx_all)
      gather_cp(0, 0).start()

    def pair_body(kk, _):
      for sub in (0, 1):
        kb = 2 * kk + sub
        base = wbase + kb * BATCH
        live = base < n_edges

        @pl.when(live)
        def _(kb=kb, base=base, sl=sub, kk=kk):
          next_ok = (wbase + (kb + 1) * BATCH) < n_edges
          if sl == 1:
            next_ok = jnp.logical_and(next_ok, kk < nhalf - 1)

          @pl.when(next_ok)
          def _():
            gather_cp(kb + 1, 1 - sl).start()

          @pl.when(kb >= 2)
          def _():
            store_cp(base - 2 * BATCH, sl).wait()

          gather_cp(kb, sl).wait()

          compute_batch(rows[sl], outs[sl])

          full = base + BATCH <= n_edges

          @pl.when(full)
          def _():
            store_cp(base, sl).start()

          @pl.when(jnp.logical_not(full))
          def _():
            def tail_body(e, _):
              @pl.when(base + e < n_edges)
              def _():
                pltpu.sync_copy(outs[sl].at[e], out_hbm.at[base + e])
              return 0

            lax.fori_loop(0, BATCH, tail_body, 0, unroll=False)

      return 0

    lax.fori_loop(0, nhalf, pair_body, 0, unroll=False)

    n_my = jnp.maximum(jnp.int32(0),
                       jnp.minimum(jnp.int32(n_edges) - wbase,
                                   jnp.int32(per_worker)))
    n_live = (n_my + BATCH - 1) // BATCH
    n_full = n_my // BATCH
    for kq in (2, 1):
      kp = n_live - kq

      @pl.when(jnp.logical_and(kp >= 0, kp < n_full))
      def _(kp=kp):
        sl = lax.rem(kp, jnp.int32(2))

        @pl.when(sl == 0)
        def _():
          store_cp(wbase + kp * BATCH, 0).wait()

        @pl.when(sl == 1)
        def _():
          store_cp(wbase + kp * BATCH, 1).wait()

  return k(xp, bflat)


def kernel(input, bonds):
  n_atoms, three, n_geoms = input.shape
  n_edges = bonds.shape[0]
  row32 = three * n_geoms // 2

  n_workers = 32
  per_worker_edges = -(-n_edges // n_workers)
  nbatch = -(-per_worker_edges // BATCH)
  nbatch += nbatch % 2
  per_worker = nbatch * BATCH
  e_pad = n_workers * per_worker

  xs = input.reshape(n_atoms, three, n_geoms // 32, 2, LANES)
  xs = jnp.swapaxes(xs, 3, 4).astype(jnp.bfloat16)
  xp = lax.bitcast_convert_type(xs, jnp.int32).reshape(n_atoms, row32)

  bflat = jnp.concatenate(
      [bonds.reshape(-1), jnp.zeros(2 * (e_pad - n_edges), jnp.int32)])

  return _run(xp, bflat, n_atoms=n_atoms, row32=row32, n_geoms=n_geoms,
              n_edges=n_edges, nbatch=nbatch, per_worker=per_worker)
